# baseline (speedup 1.0000x reference)
import functools

import jax
import jax.numpy as jnp
from jax import lax
from jax.experimental import pallas as pl
from jax.experimental.pallas import tpu as pltpu
from jax.experimental.pallas import tpu_sc as plsc

_COORDS = ((0, 2, 3), (1, 1, 2), (2, 1, 4), (3, 5, 1))
_D1, _D2 = 2048, 2048
_ROW_W = 128
_L = 16

_LINEAR = [a * _D1 * _D2 + b * _D2 + c for (a, b, c) in _COORDS]
_ROWS = [l // _ROW_W for l in _LINEAR]
_OFFS = [l % _ROW_W for l in _LINEAR]
_ROW_IDX_LIST = _ROWS + [_ROWS[0]] * (_L - len(_ROWS))


def _sc_body(flat_hbm, idx_hbm, out_hbm, idx_v, rows_v, out_v, sem):
    wid = lax.axis_index("s") * 2 + lax.axis_index("c")

    @pl.when(wid == 0)
    def _():
        pltpu.sync_copy(idx_hbm, idx_v)
        pltpu.async_copy(flat_hbm.at[idx_v], rows_v, sem).wait()
        dnums = lax.GatherDimensionNumbers(
            offset_dims=(), collapsed_slice_dims=(0,), start_index_map=(0,))
        total = jnp.zeros((_L,), jnp.float32)
        for j, off in enumerate(_OFFS):
            sub = (off // _L) * _L
            window = rows_v[j, pl.ds(sub, _L)]
            bidx = jnp.full((_L, 1), off - sub, dtype=jnp.int32)
            total = total + lax.gather(
                window, bidx, dnums, slice_sizes=(1,),
                mode=lax.GatherScatterMode.PROMISE_IN_BOUNDS)
        out_v[...] = total
        pltpu.sync_copy(out_v, out_hbm)


def kernel(values):
    flat = values.reshape(-1, _ROW_W)
    mesh = plsc.VectorSubcoreMesh(core_axis_name="c", subcore_axis_name="s")
    k = functools.partial(
        pl.kernel,
        mesh=mesh,
        out_type=jax.ShapeDtypeStruct((_L,), jnp.float32),
        scratch_types=[
            pltpu.VMEM((_L,), jnp.int32),
            pltpu.VMEM((_L, _ROW_W), jnp.float32),
            pltpu.VMEM((_L,), jnp.float32),
            pltpu.SemaphoreType.DMA,
        ],
    )(_sc_body)
    out = k(flat, jnp.array(_ROW_IDX_LIST, dtype=jnp.int32))
    return out[0]


# speedup vs baseline: 1.0339x; 1.0339x over previous
import functools

import jax
import jax.numpy as jnp
from jax import lax
from jax.experimental import pallas as pl
from jax.experimental.pallas import tpu as pltpu
from jax.experimental.pallas import tpu_sc as plsc

_COORDS = ((0, 2, 3), (1, 1, 2), (2, 1, 4), (3, 5, 1))
_D1, _D2 = 2048, 2048
_ROW_W = 128
_L = 16

_LINEAR = [a * _D1 * _D2 + b * _D2 + c for (a, b, c) in _COORDS]
_ROWS = [l // _ROW_W for l in _LINEAR]
_OFFS = [l % _ROW_W for l in _LINEAR]


def _sc_body(flat_hbm, out_hbm, rows_v, out_v, sem):
    wid = lax.axis_index("s") * 2 + lax.axis_index("c")

    @pl.when(wid == 0)
    def _():
        lane = lax.iota(jnp.int32, _L)
        idx = jnp.full((_L,), _ROWS[0], dtype=jnp.int32)
        for j, r in enumerate(_ROWS[1:], start=1):
            idx = jnp.where(lane == j, r, idx)
        pltpu.async_copy(flat_hbm.at[idx], rows_v, sem).wait()
        dnums = lax.GatherDimensionNumbers(
            offset_dims=(), collapsed_slice_dims=(0,), start_index_map=(0,))
        total = jnp.zeros((_L,), jnp.float32)
        for j, off in enumerate(_OFFS):
            sub = (off // _L) * _L
            window = rows_v[j, pl.ds(sub, _L)]
            bidx = jnp.full((_L, 1), off - sub, dtype=jnp.int32)
            total = total + lax.gather(
                window, bidx, dnums, slice_sizes=(1,),
                mode=lax.GatherScatterMode.PROMISE_IN_BOUNDS)
        out_v[...] = total
        pltpu.sync_copy(out_v, out_hbm)


def kernel(values):
    flat = values.reshape(-1, _ROW_W)
    mesh = plsc.VectorSubcoreMesh(
        core_axis_name="c", subcore_axis_name="s", num_cores=1)
    k = functools.partial(
        pl.kernel,
        mesh=mesh,
        out_type=jax.ShapeDtypeStruct((_L,), jnp.float32),
        scratch_types=[
            pltpu.VMEM((_L, _ROW_W), jnp.float32),
            pltpu.VMEM((_L,), jnp.float32),
            pltpu.SemaphoreType.DMA,
        ],
    )(_sc_body)
    out = k(flat)
    return out[0]


# speedup vs baseline: 193.2097x; 186.8729x over previous
import jax
import jax.numpy as jnp
from jax import lax
from jax.experimental import pallas as pl

_COORDS = ((0, 2, 3), (1, 1, 2), (2, 1, 4), (3, 5, 1))


def _body(x_ref, o_ref):
    x = x_ref[...]
    i = lax.broadcasted_iota(jnp.int32, x.shape, 0)
    j = lax.broadcasted_iota(jnp.int32, x.shape, 1)
    k = lax.broadcasted_iota(jnp.int32, x.shape, 2)
    mask = None
    for (a, b, c) in _COORDS:
        m = (i == a) & (j == b) & (k == c)
        mask = m if mask is None else (mask | m)
    o_ref[...] = jnp.sum(jnp.where(mask, x, 0.0)).reshape(1, 1)


def kernel(values):
    out = pl.pallas_call(
        _body,
        out_shape=jax.ShapeDtypeStruct((1, 1), jnp.float32),
        grid=(1,),
        in_specs=[pl.BlockSpec((4, 8, 128), lambda i: (0, 0, 0))],
        out_specs=pl.BlockSpec((1, 1), lambda i: (0, 0)),
    )(values)
    return out[0, 0]


# speedup vs baseline: 197.8409x; 1.0240x over previous
import jax
import jax.numpy as jnp
from jax import lax
from jax.experimental import pallas as pl
from jax.experimental.pallas import tpu as pltpu

_COORDS = ((0, 2, 3), (1, 1, 2), (2, 1, 4), (3, 5, 1))


def _body(x_ref, o_ref):
    x = x_ref[...]
    i = lax.broadcasted_iota(jnp.int32, x.shape, 0)
    j = lax.broadcasted_iota(jnp.int32, x.shape, 1)
    k = lax.broadcasted_iota(jnp.int32, x.shape, 2)
    mask = None
    for (a, b, c) in _COORDS:
        m = (i == a) & (j == b) & (k == c)
        mask = m if mask is None else (mask | m)
    o_ref[0, 0] = jnp.sum(jnp.where(mask, x, 0.0))


def kernel(values):
    out = pl.pallas_call(
        _body,
        out_shape=jax.ShapeDtypeStruct((1, 1), jnp.float32),
        grid=(1,),
        in_specs=[pl.BlockSpec((4, 8, 128), lambda i: (0, 0, 0))],
        out_specs=pl.BlockSpec(memory_space=pltpu.SMEM),
    )(values)
    return out[0, 0]
